# Initial kernel scaffold; baseline (speedup 1.0000x reference)
import jax
import jax.numpy as jnp
from jax.experimental import pallas as pl


def kernel(feat, edge_index, etypes, W_e, b_e, W_ih, W_hh, b_ih, b_hh, out_W, out_b):
    raise NotImplementedError("write your pallas kernel here")


# speedup vs baseline: 4.3587x; 4.3587x over previous
import functools

import jax
import jax.numpy as jnp
from jax.experimental import pallas as pl
from jax.experimental.pallas import tpu as pltpu
from jax.experimental.pallas import tpu_sc as plsc

F = 128
NT = 3
NSTEP = 3
N_NODES = 10000
N_EDGES = 320000
NPAD = 10240

NC = 2
NS = 16
NW = NC * NS
EPT = N_EDGES // NW
K = 80
NCHUNK = EPT // K
ROWS_PER_TILE = NPAD // NS
BN = 256
GRID = NPAD // BN


def _pre_body(h_ref, we_ref, be_ref, whh_ref, bhh_ref, hcat_ref, gh_ref):
    x = h_ref[...]
    for i in range(NT):
        hcat_ref[i] = jax.lax.dot_general(
            x, we_ref[i], (((1,), (1,)), ((), ())),
            preferred_element_type=jnp.float32) + be_ref[i]
    gh_ref[...] = jax.lax.dot_general(
        x, whh_ref[...], (((1,), (1,)), ((), ())),
        preferred_element_type=jnp.float32) + bhh_ref[...]


_pre_call = pl.pallas_call(
    _pre_body,
    grid=(GRID,),
    in_specs=[
        pl.BlockSpec((BN, F), lambda i: (i, 0)),
        pl.BlockSpec((NT, F, F), lambda i: (0, 0, 0)),
        pl.BlockSpec((NT, 1, F), lambda i: (0, 0, 0)),
        pl.BlockSpec((3 * F, F), lambda i: (0, 0)),
        pl.BlockSpec((1, 3 * F), lambda i: (0, 0)),
    ],
    out_specs=[
        pl.BlockSpec((NT, BN, F), lambda i: (0, i, 0)),
        pl.BlockSpec((BN, 3 * F), lambda i: (i, 0)),
    ],
    out_shape=[
        jax.ShapeDtypeStruct((NT, NPAD, F), jnp.float32),
        jax.ShapeDtypeStruct((NPAD, 3 * F), jnp.float32),
    ],
)


def _gru_body(ap_ref, gh_ref, h_ref, wih_ref, bih_ref, hnew_ref):
    a = ap_ref[0] + ap_ref[1]
    gi = jax.lax.dot_general(
        a, wih_ref[...], (((1,), (1,)), ((), ())),
        preferred_element_type=jnp.float32) + bih_ref[...]
    gh = gh_ref[...]
    h = h_ref[...]
    r = jax.nn.sigmoid(gi[:, :F] + gh[:, :F])
    z = jax.nn.sigmoid(gi[:, F:2 * F] + gh[:, F:2 * F])
    n = jnp.tanh(gi[:, 2 * F:] + r * gh[:, 2 * F:])
    hnew_ref[...] = (1.0 - z) * n + z * h


_gru_call = pl.pallas_call(
    _gru_body,
    grid=(GRID,),
    in_specs=[
        pl.BlockSpec((NC, BN, F), lambda i: (0, i, 0)),
        pl.BlockSpec((BN, 3 * F), lambda i: (i, 0)),
        pl.BlockSpec((BN, F), lambda i: (i, 0)),
        pl.BlockSpec((3 * F, F), lambda i: (0, 0)),
        pl.BlockSpec((1, 3 * F), lambda i: (0, 0)),
    ],
    out_specs=pl.BlockSpec((BN, F), lambda i: (i, 0)),
    out_shape=jax.ShapeDtypeStruct((NPAD, F), jnp.float32),
)


def _head_body(h_ref, f_ref, w_ref, b_ref, o_ref):
    hf = jnp.concatenate([h_ref[...], f_ref[...]], axis=1)
    o_ref[...] = jax.nn.sigmoid(
        jax.lax.dot_general(hf, w_ref[...], (((1,), (1,)), ((), ())),
                            preferred_element_type=jnp.float32) + b_ref[0, 0])


_head_call = pl.pallas_call(
    _head_body,
    grid=(GRID,),
    in_specs=[
        pl.BlockSpec((BN, F), lambda i: (i, 0)),
        pl.BlockSpec((BN, F), lambda i: (i, 0)),
        pl.BlockSpec((F, 2 * F), lambda i: (0, 0)),
        pl.BlockSpec((1, 1), lambda i: (0, 0)),
    ],
    out_specs=pl.BlockSpec((BN, F), lambda i: (i, 0)),
    out_shape=jax.ShapeDtypeStruct((NPAD, F), jnp.float32),
)


def _sc_agg_body(hcat, src, et, dst, out, src_v, et_v, dst_v, cidx_v, rows_v,
                 acc, sem):
    cid = jax.lax.axis_index("c")
    sid = jax.lax.axis_index("s")
    wid = cid * NS + sid

    zero16 = jnp.zeros((16,), jnp.float32)

    def _zr(i, carry):
        rows_v[i // (F // 16), pl.ds((i % (F // 16)) * 16, 16)] = zero16
        return carry

    jax.lax.fori_loop(0, K * (F // 16), _zr, 0)

    def _zacc(kk, carry):
        pltpu.sync_copy(rows_v, acc.at[pl.ds(sid * ROWS_PER_TILE + kk * K, K)])
        return carry

    jax.lax.fori_loop(0, ROWS_PER_TILE // K, _zacc, 0)
    plsc.subcore_barrier()

    ebase = wid * EPT

    def _chunk(j, carry):
        eo = ebase + j * K
        pltpu.sync_copy(src.at[pl.ds(eo, K)], src_v)
        pltpu.sync_copy(et.at[pl.ds(eo, K)], et_v)
        pltpu.sync_copy(dst.at[pl.ds(eo, K)], dst_v)
        for q in range(K // 16):
            sl = pl.ds(q * 16, 16)
            cidx_v[sl] = src_v[sl] + et_v[sl] * NPAD
        pltpu.async_copy(hcat.at[cidx_v], rows_v, sem).wait()
        pltpu.sync_copy(rows_v, acc.at[dst_v], add=True)
        return carry

    jax.lax.fori_loop(0, NCHUNK, _chunk, 0)
    plsc.subcore_barrier()

    def _wb(kk, carry):
        r0 = sid * ROWS_PER_TILE + kk * K
        pltpu.sync_copy(acc.at[pl.ds(r0, K)], rows_v)
        pltpu.sync_copy(rows_v, out.at[cid, pl.ds(r0, K)])
        return carry

    jax.lax.fori_loop(0, ROWS_PER_TILE // K, _wb, 0)


@functools.cache
def _sc_agg_call():
    return pl.kernel(
        _sc_agg_body,
        out_type=jax.ShapeDtypeStruct((NC, NPAD, F), jnp.float32),
        mesh=plsc.VectorSubcoreMesh(core_axis_name="c", subcore_axis_name="s",
                                    num_cores=NC, num_subcores=NS),
        scratch_types=[
            pltpu.VMEM((K,), jnp.int32),
            pltpu.VMEM((K,), jnp.int32),
            pltpu.VMEM((K,), jnp.int32),
            pltpu.VMEM((K,), jnp.int32),
            pltpu.VMEM((K, F), jnp.float32),
            pltpu.VMEM_SHARED((NPAD, F), jnp.float32),
            pltpu.SemaphoreType.DMA,
        ],
    )


def kernel(feat, edge_index, etypes, W_e, b_e, W_ih, W_hh, b_ih, b_hh,
           out_W, out_b):
    src = edge_index[0]
    dst = edge_index[1]
    be3 = b_e.reshape(NT, 1, F)
    bhh2 = b_hh.reshape(1, 3 * F)
    bih2 = b_ih.reshape(1, 3 * F)
    ob2 = out_b.reshape(1, 1)
    fpad = jnp.pad(feat, ((0, NPAD - N_NODES), (0, 0)))

    h = fpad
    for _ in range(NSTEP):
        hcat, gh = _pre_call(h, W_e, be3, W_hh, bhh2)
        apart = _sc_agg_call()(hcat.reshape(NT * NPAD, F), src, etypes, dst)
        h = _gru_call(apart, gh, h, W_ih, bih2)
    wpad = jnp.pad(out_W, ((0, F - 1), (0, 0)))
    out = _head_call(h, fpad, wpad, ob2)
    return out[:N_NODES, 0]


# speedup vs baseline: 9.4618x; 2.1708x over previous
import functools

import jax
import jax.numpy as jnp
from jax.experimental import pallas as pl
from jax.experimental.pallas import tpu as pltpu
from jax.experimental.pallas import tpu_sc as plsc

F = 128
NT = 3
NSTEP = 3
N_NODES = 10000
N_EDGES = 320000
NPAD = 10240

NC = 2
NS = 16
NW = NC * NS
EPT = N_EDGES // NW
K = 80
NCHUNK = EPT // K
ROWS_PER_TILE = NPAD // NS
BN = 256
GRID = NPAD // BN


def _pre_body(h_ref, we_ref, be_ref, whh_ref, bhh_ref, hcat_ref, gh_ref):
    x = h_ref[...]
    for i in range(NT):
        hcat_ref[i] = jax.lax.dot_general(
            x, we_ref[i], (((1,), (1,)), ((), ())),
            preferred_element_type=jnp.float32) + be_ref[i]
    gh_ref[...] = jax.lax.dot_general(
        x, whh_ref[...], (((1,), (1,)), ((), ())),
        preferred_element_type=jnp.float32) + bhh_ref[...]


_pre_call = pl.pallas_call(
    _pre_body,
    grid=(GRID,),
    in_specs=[
        pl.BlockSpec((BN, F), lambda i: (i, 0)),
        pl.BlockSpec((NT, F, F), lambda i: (0, 0, 0)),
        pl.BlockSpec((NT, 1, F), lambda i: (0, 0, 0)),
        pl.BlockSpec((3 * F, F), lambda i: (0, 0)),
        pl.BlockSpec((1, 3 * F), lambda i: (0, 0)),
    ],
    out_specs=[
        pl.BlockSpec((NT, BN, F), lambda i: (0, i, 0)),
        pl.BlockSpec((BN, 3 * F), lambda i: (i, 0)),
    ],
    out_shape=[
        jax.ShapeDtypeStruct((NT, NPAD, F), jnp.float32),
        jax.ShapeDtypeStruct((NPAD, 3 * F), jnp.float32),
    ],
)


def _gru_body(ap_ref, gh_ref, h_ref, wih_ref, bih_ref, hnew_ref):
    a = ap_ref[0] + ap_ref[1]
    gi = jax.lax.dot_general(
        a, wih_ref[...], (((1,), (1,)), ((), ())),
        preferred_element_type=jnp.float32) + bih_ref[...]
    gh = gh_ref[...]
    h = h_ref[...]
    r = jax.nn.sigmoid(gi[:, :F] + gh[:, :F])
    z = jax.nn.sigmoid(gi[:, F:2 * F] + gh[:, F:2 * F])
    n = jnp.tanh(gi[:, 2 * F:] + r * gh[:, 2 * F:])
    hnew_ref[...] = (1.0 - z) * n + z * h


_gru_call = pl.pallas_call(
    _gru_body,
    grid=(GRID,),
    in_specs=[
        pl.BlockSpec((NC, BN, F), lambda i: (0, i, 0)),
        pl.BlockSpec((BN, 3 * F), lambda i: (i, 0)),
        pl.BlockSpec((BN, F), lambda i: (i, 0)),
        pl.BlockSpec((3 * F, F), lambda i: (0, 0)),
        pl.BlockSpec((1, 3 * F), lambda i: (0, 0)),
    ],
    out_specs=pl.BlockSpec((BN, F), lambda i: (i, 0)),
    out_shape=jax.ShapeDtypeStruct((NPAD, F), jnp.float32),
)


def _head_body(h_ref, f_ref, w_ref, b_ref, o_ref):
    hf = jnp.concatenate([h_ref[...], f_ref[...]], axis=1)
    o_ref[...] = jax.nn.sigmoid(
        jax.lax.dot_general(hf, w_ref[...], (((1,), (1,)), ((), ())),
                            preferred_element_type=jnp.float32) + b_ref[0, 0])


_head_call = pl.pallas_call(
    _head_body,
    grid=(GRID,),
    in_specs=[
        pl.BlockSpec((BN, F), lambda i: (i, 0)),
        pl.BlockSpec((BN, F), lambda i: (i, 0)),
        pl.BlockSpec((F, 2 * F), lambda i: (0, 0)),
        pl.BlockSpec((1, 1), lambda i: (0, 0)),
    ],
    out_specs=pl.BlockSpec((BN, F), lambda i: (i, 0)),
    out_shape=jax.ShapeDtypeStruct((NPAD, F), jnp.float32),
)


def _pack_body(s_ref, e_ref, d_ref, p_ref):
    p_ref[...] = (s_ref[...] + e_ref[...] * NPAD) + d_ref[...] * 65536


_pack_call = pl.pallas_call(
    _pack_body,
    in_specs=[pl.BlockSpec((N_EDGES // F, F), lambda: (0, 0))] * 3,
    out_specs=pl.BlockSpec((N_EDGES // F, F), lambda: (0, 0)),
    out_shape=jax.ShapeDtypeStruct((N_EDGES // F, F), jnp.int32),
)


def _sc_agg_body(hcat, packed, out, pk2, rows0, rows1, ci0, ci1, ds0, ds1,
                 acc, sem0, sem1):
    cid = jax.lax.axis_index("c")
    sid = jax.lax.axis_index("s")
    wid = cid * NS + sid

    ldp = pltpu.async_copy(packed.at[wid], pk2, sem0)

    zero16 = jnp.zeros((16,), jnp.float32)

    def _zr(i, carry):
        rows0[i // (F // 16), pl.ds((i % (F // 16)) * 16, 16)] = zero16
        return carry

    jax.lax.fori_loop(0, K * (F // 16), _zr, 0)

    def _zacc(kk, carry):
        pltpu.sync_copy(rows0, acc.at[pl.ds(sid * ROWS_PER_TILE + kk * K, K)])
        return carry

    jax.lax.fori_loop(0, ROWS_PER_TILE // K, _zacc, 0)
    ldp.wait()

    cbufs = ((ci0, ds0), (ci1, ds1))

    def _unpack(j, b):
        for q in range(K // 16):
            sl = pl.ds(q * 16, 16)
            p = pk2[j, sl]
            cbufs[b][0][sl] = jnp.bitwise_and(p, 65535)
            cbufs[b][1][sl] = jax.lax.shift_right_logical(p, 16)

    _unpack(0, 0)
    pltpu.async_copy(hcat.at[ci0], rows0, sem0)
    plsc.subcore_barrier()

    bufs = ((rows0, sem0), (rows1, sem1))

    def _step(j, b):
        rows, sem = bufs[b]
        nrows, nsem = bufs[1 - b]

        @pl.when(j + 1 < NCHUNK)
        def _():
            _unpack(j + 1, 1 - b)
            pltpu.async_copy(hcat.at[cbufs[1 - b][0]], nrows, nsem)

        pltpu.make_async_copy(hcat.at[cbufs[b][0]], rows, sem).wait()
        pltpu.sync_copy(rows, acc.at[cbufs[b][1]], add=True)

    def _pair(jj, carry):
        for b in range(2):
            _step(jj * 2 + b, b)
        return carry

    jax.lax.fori_loop(0, NCHUNK // 2, _pair, 0)
    if NCHUNK % 2:
        _step(NCHUNK - 1, 0)

    plsc.subcore_barrier()

    def _wb(kk, carry):
        r0 = sid * ROWS_PER_TILE + kk * K
        pltpu.sync_copy(acc.at[pl.ds(r0, K)], rows0)
        pltpu.sync_copy(rows0, out.at[cid, pl.ds(r0, K)])
        return carry

    jax.lax.fori_loop(0, ROWS_PER_TILE // K, _wb, 0)


@functools.cache
def _sc_agg_call():
    return pl.kernel(
        _sc_agg_body,
        out_type=jax.ShapeDtypeStruct((NC, NPAD, F), jnp.float32),
        mesh=plsc.VectorSubcoreMesh(core_axis_name="c", subcore_axis_name="s",
                                    num_cores=NC, num_subcores=NS),
        scratch_types=[
            pltpu.VMEM((NCHUNK, K), jnp.int32),
            pltpu.VMEM((K, F), jnp.float32),
            pltpu.VMEM((K, F), jnp.float32),
            pltpu.VMEM((K,), jnp.int32),
            pltpu.VMEM((K,), jnp.int32),
            pltpu.VMEM((K,), jnp.int32),
            pltpu.VMEM((K,), jnp.int32),
            pltpu.VMEM_SHARED((NPAD, F), jnp.float32),
            pltpu.SemaphoreType.DMA,
            pltpu.SemaphoreType.DMA,
        ],
    )


def kernel(feat, edge_index, etypes, W_e, b_e, W_ih, W_hh, b_ih, b_hh,
           out_W, out_b):
    src = edge_index[0]
    dst = edge_index[1]
    be3 = b_e.reshape(NT, 1, F)
    bhh2 = b_hh.reshape(1, 3 * F)
    bih2 = b_ih.reshape(1, 3 * F)
    ob2 = out_b.reshape(1, 1)
    fpad = jnp.pad(feat, ((0, NPAD - N_NODES), (0, 0)))
    packed = _pack_call(src.reshape(N_EDGES // F, F),
                        etypes.reshape(N_EDGES // F, F),
                        dst.reshape(N_EDGES // F, F)).reshape(NW, NCHUNK, K)

    h = fpad
    for _ in range(NSTEP):
        hcat, gh = _pre_call(h, W_e, be3, W_hh, bhh2)
        apart = _sc_agg_call()(hcat.reshape(NT * NPAD, F), packed)
        h = _gru_call(apart, gh, h, W_ih, bih2)
    wpad = jnp.pad(out_W, ((0, F - 1), (0, 0)))
    out = _head_call(h, fpad, wpad, ob2)
    return out[:N_NODES, 0]


# speedup vs baseline: 10.5661x; 1.1167x over previous
import functools

import jax
import jax.numpy as jnp
from jax.experimental import pallas as pl
from jax.experimental.pallas import tpu as pltpu
from jax.experimental.pallas import tpu_sc as plsc

F = 128
NT = 3
NSTEP = 3
N_NODES = 10000
N_EDGES = 320000
NPAD = 10240

NC = 2
NS = 16
NW = NC * NS
EPT = N_EDGES // NW
K = 80
NCHUNK = EPT // K
ROWS_PER_TILE = NPAD // NS
BN = 256
GRID = NPAD // BN


def _pre_body(h_ref, we_ref, be_ref, whh_ref, bhh_ref, hcat_ref, gh_ref):
    x = h_ref[...]
    for i in range(NT):
        hcat_ref[i] = jax.lax.dot_general(
            x, we_ref[i], (((1,), (1,)), ((), ())),
            preferred_element_type=jnp.float32) + be_ref[i]
    gh_ref[...] = jax.lax.dot_general(
        x, whh_ref[...], (((1,), (1,)), ((), ())),
        preferred_element_type=jnp.float32) + bhh_ref[...]


_pre_call = pl.pallas_call(
    _pre_body,
    grid=(GRID,),
    in_specs=[
        pl.BlockSpec((BN, F), lambda i: (i, 0)),
        pl.BlockSpec((NT, F, F), lambda i: (0, 0, 0)),
        pl.BlockSpec((NT, 1, F), lambda i: (0, 0, 0)),
        pl.BlockSpec((3 * F, F), lambda i: (0, 0)),
        pl.BlockSpec((1, 3 * F), lambda i: (0, 0)),
    ],
    out_specs=[
        pl.BlockSpec((NT, BN, F), lambda i: (0, i, 0)),
        pl.BlockSpec((BN, 3 * F), lambda i: (i, 0)),
    ],
    out_shape=[
        jax.ShapeDtypeStruct((NT, NPAD, F), jnp.float32),
        jax.ShapeDtypeStruct((NPAD, 3 * F), jnp.float32),
    ],
)


def _gru_math(ap_ref, gh_ref, h_ref, wih_ref, bih_ref):
    a = ap_ref[0] + ap_ref[1]
    gi = jax.lax.dot_general(
        a, wih_ref[...], (((1,), (1,)), ((), ())),
        preferred_element_type=jnp.float32) + bih_ref[...]
    gh = gh_ref[...]
    h = h_ref[...]
    r = jax.nn.sigmoid(gi[:, :F] + gh[:, :F])
    z = jax.nn.sigmoid(gi[:, F:2 * F] + gh[:, F:2 * F])
    n = jnp.tanh(gi[:, 2 * F:] + r * gh[:, 2 * F:])
    return (1.0 - z) * n + z * h


def _gru_pre_body(ap_ref, gh_ref, h_ref, wih_ref, bih_ref, we_ref, be_ref,
                  whh_ref, bhh_ref, hnew_ref, hcat_ref, ghn_ref):
    hn = _gru_math(ap_ref, gh_ref, h_ref, wih_ref, bih_ref)
    hnew_ref[...] = hn
    for i in range(NT):
        hcat_ref[i] = jax.lax.dot_general(
            hn, we_ref[i], (((1,), (1,)), ((), ())),
            preferred_element_type=jnp.float32) + be_ref[i]
    ghn_ref[...] = jax.lax.dot_general(
        hn, whh_ref[...], (((1,), (1,)), ((), ())),
        preferred_element_type=jnp.float32) + bhh_ref[...]


_gru_pre_call = pl.pallas_call(
    _gru_pre_body,
    grid=(GRID,),
    in_specs=[
        pl.BlockSpec((NC, BN, F), lambda i: (0, i, 0)),
        pl.BlockSpec((BN, 3 * F), lambda i: (i, 0)),
        pl.BlockSpec((BN, F), lambda i: (i, 0)),
        pl.BlockSpec((3 * F, F), lambda i: (0, 0)),
        pl.BlockSpec((1, 3 * F), lambda i: (0, 0)),
        pl.BlockSpec((NT, F, F), lambda i: (0, 0, 0)),
        pl.BlockSpec((NT, 1, F), lambda i: (0, 0, 0)),
        pl.BlockSpec((3 * F, F), lambda i: (0, 0)),
        pl.BlockSpec((1, 3 * F), lambda i: (0, 0)),
    ],
    out_specs=[
        pl.BlockSpec((BN, F), lambda i: (i, 0)),
        pl.BlockSpec((NT, BN, F), lambda i: (0, i, 0)),
        pl.BlockSpec((BN, 3 * F), lambda i: (i, 0)),
    ],
    out_shape=[
        jax.ShapeDtypeStruct((NPAD, F), jnp.float32),
        jax.ShapeDtypeStruct((NT, NPAD, F), jnp.float32),
        jax.ShapeDtypeStruct((NPAD, 3 * F), jnp.float32),
    ],
)


def _gru_head_body(ap_ref, gh_ref, h_ref, wih_ref, bih_ref, f_ref, w_ref,
                   b_ref, o_ref):
    hn = _gru_math(ap_ref, gh_ref, h_ref, wih_ref, bih_ref)
    hf = jnp.concatenate([hn, f_ref[...]], axis=1)
    o_ref[...] = jax.nn.sigmoid(
        jax.lax.dot_general(hf, w_ref[...], (((1,), (1,)), ((), ())),
                            preferred_element_type=jnp.float32) + b_ref[0, 0])


_gru_head_call = pl.pallas_call(
    _gru_head_body,
    grid=(GRID,),
    in_specs=[
        pl.BlockSpec((NC, BN, F), lambda i: (0, i, 0)),
        pl.BlockSpec((BN, 3 * F), lambda i: (i, 0)),
        pl.BlockSpec((BN, F), lambda i: (i, 0)),
        pl.BlockSpec((3 * F, F), lambda i: (0, 0)),
        pl.BlockSpec((1, 3 * F), lambda i: (0, 0)),
        pl.BlockSpec((BN, F), lambda i: (i, 0)),
        pl.BlockSpec((F, 2 * F), lambda i: (0, 0)),
        pl.BlockSpec((1, 1), lambda i: (0, 0)),
    ],
    out_specs=pl.BlockSpec((BN, F), lambda i: (i, 0)),
    out_shape=jax.ShapeDtypeStruct((NPAD, F), jnp.float32),
)


def _pack_body(s_ref, e_ref, d_ref, p_ref):
    p_ref[...] = (s_ref[...] + e_ref[...] * NPAD) + d_ref[...] * 65536


_pack_call = pl.pallas_call(
    _pack_body,
    in_specs=[pl.BlockSpec((N_EDGES // F, F), lambda: (0, 0))] * 3,
    out_specs=pl.BlockSpec((N_EDGES // F, F), lambda: (0, 0)),
    out_shape=jax.ShapeDtypeStruct((N_EDGES // F, F), jnp.int32),
)


def _sc_agg_body(hcat, packed, out, pk2, rows0, rows1, ci0, ci1, ds0, ds1,
                 acc, sem0, sem1, ses0, ses1):
    cid = jax.lax.axis_index("c")
    sid = jax.lax.axis_index("s")
    wid = cid * NS + sid

    ldp = pltpu.async_copy(packed.at[wid], pk2, sem0)

    zero16 = jnp.zeros((16,), jnp.float32)

    def _zr(i, carry):
        rows0[i // (F // 16), pl.ds((i % (F // 16)) * 16, 16)] = zero16
        return carry

    jax.lax.fori_loop(0, K * (F // 16), _zr, 0)

    def _zacc(kk, carry):
        pltpu.sync_copy(rows0, acc.at[pl.ds(sid * ROWS_PER_TILE + kk * K, K)])
        return carry

    jax.lax.fori_loop(0, ROWS_PER_TILE // K, _zacc, 0)
    ldp.wait()

    cbufs = ((ci0, ds0), (ci1, ds1))

    def _unpack(j, b):
        for q in range(K // 16):
            sl = pl.ds(q * 16, 16)
            p = pk2[j, sl]
            cbufs[b][0][sl] = jnp.bitwise_and(p, 65535)
            cbufs[b][1][sl] = jax.lax.shift_right_logical(p, 16)

    _unpack(0, 0)
    pltpu.async_copy(hcat.at[ci0], rows0, sem0)
    plsc.subcore_barrier()

    bufs = ((rows0, sem0, ses0), (rows1, sem1, ses1))

    def _step(j, b):
        rows, semg, sems = bufs[b]
        nrows, nsemg, nsems = bufs[1 - b]

        @pl.when(jnp.logical_and(j + 1 < NCHUNK, j >= 1))
        def _():
            pltpu.make_async_copy(nrows, acc.at[cbufs[1 - b][1]], nsems).wait()

        @pl.when(j + 1 < NCHUNK)
        def _():
            _unpack(j + 1, 1 - b)
            pltpu.async_copy(hcat.at[cbufs[1 - b][0]], nrows, nsemg)

        pltpu.make_async_copy(hcat.at[cbufs[b][0]], rows, semg).wait()
        pltpu.async_copy(rows, acc.at[cbufs[b][1]], sems, add=True)

    def _pair(jj, carry):
        for b in range(2):
            _step(jj * 2 + b, b)
        return carry

    jax.lax.fori_loop(0, NCHUNK // 2, _pair, 0)
    if NCHUNK % 2:
        _step(NCHUNK - 1, 0)
    pltpu.make_async_copy(rows1, acc.at[ds1], ses1).wait()
    pltpu.make_async_copy(rows0, acc.at[ds0], ses0).wait()

    plsc.subcore_barrier()

    def _wb(kk, carry):
        r0 = sid * ROWS_PER_TILE + kk * K
        pltpu.sync_copy(acc.at[pl.ds(r0, K)], rows0)
        pltpu.sync_copy(rows0, out.at[cid, pl.ds(r0, K)])
        return carry

    jax.lax.fori_loop(0, ROWS_PER_TILE // K, _wb, 0)


@functools.cache
def _sc_agg_call():
    return pl.kernel(
        _sc_agg_body,
        out_type=jax.ShapeDtypeStruct((NC, NPAD, F), jnp.float32),
        mesh=plsc.VectorSubcoreMesh(core_axis_name="c", subcore_axis_name="s",
                                    num_cores=NC, num_subcores=NS),
        scratch_types=[
            pltpu.VMEM((NCHUNK, K), jnp.int32),
            pltpu.VMEM((K, F), jnp.float32),
            pltpu.VMEM((K, F), jnp.float32),
            pltpu.VMEM((K,), jnp.int32),
            pltpu.VMEM((K,), jnp.int32),
            pltpu.VMEM((K,), jnp.int32),
            pltpu.VMEM((K,), jnp.int32),
            pltpu.VMEM_SHARED((NPAD, F), jnp.float32),
            pltpu.SemaphoreType.DMA,
            pltpu.SemaphoreType.DMA,
            pltpu.SemaphoreType.DMA,
            pltpu.SemaphoreType.DMA,
        ],
    )


def kernel(feat, edge_index, etypes, W_e, b_e, W_ih, W_hh, b_ih, b_hh,
           out_W, out_b):
    src = edge_index[0]
    dst = edge_index[1]
    be3 = b_e.reshape(NT, 1, F)
    bhh2 = b_hh.reshape(1, 3 * F)
    bih2 = b_ih.reshape(1, 3 * F)
    ob2 = out_b.reshape(1, 1)
    fpad = jnp.pad(feat, ((0, NPAD - N_NODES), (0, 0)))
    packed = _pack_call(src.reshape(N_EDGES // F, F),
                        etypes.reshape(N_EDGES // F, F),
                        dst.reshape(N_EDGES // F, F)).reshape(NW, NCHUNK, K)

    h = fpad
    hcat, gh = _pre_call(h, W_e, be3, W_hh, bhh2)
    for _ in range(NSTEP - 1):
        apart = _sc_agg_call()(hcat.reshape(NT * NPAD, F), packed)
        h, hcat, gh = _gru_pre_call(apart, gh, h, W_ih, bih2, W_e, be3,
                                    W_hh, bhh2)
    apart = _sc_agg_call()(hcat.reshape(NT * NPAD, F), packed)
    wpad = jnp.pad(out_W, ((0, F - 1), (0, 0)))
    out = _gru_head_call(apart, gh, h, W_ih, bih2, fpad, wpad, ob2)
    return out[:N_NODES, 0]


# speedup vs baseline: 10.9028x; 1.0319x over previous
import functools

import jax
import jax.numpy as jnp
from jax.experimental import pallas as pl
from jax.experimental.pallas import tpu as pltpu
from jax.experimental.pallas import tpu_sc as plsc

F = 128
NT = 3
NSTEP = 3
N_NODES = 10000
N_EDGES = 320000
NPAD = 10240

NC = 2
NS = 16
NW = NC * NS
EPT = N_EDGES // NW
K = 80
NCHUNK = EPT // K
ROWS_PER_TILE = NPAD // NS
BN = 256
GRID = NPAD // BN


def _pre_body(h_ref, we_ref, be_ref, hcat_ref):
    x = h_ref[...]
    for i in range(NT):
        hcat_ref[i] = jax.lax.dot_general(
            x, we_ref[i], (((1,), (1,)), ((), ())),
            preferred_element_type=jnp.float32) + be_ref[i]


_pre_call = pl.pallas_call(
    _pre_body,
    grid=(GRID,),
    in_specs=[
        pl.BlockSpec((BN, F), lambda i: (i, 0)),
        pl.BlockSpec((NT, F, F), lambda i: (0, 0, 0)),
        pl.BlockSpec((NT, 1, F), lambda i: (0, 0, 0)),
    ],
    out_specs=pl.BlockSpec((NT, BN, F), lambda i: (0, i, 0)),
    out_shape=jax.ShapeDtypeStruct((NT, NPAD, F), jnp.float32),
)


def _gru_math(ap_ref, h_ref, wih_ref, bih_ref, whh_ref, bhh_ref):
    a = ap_ref[0] + ap_ref[1]
    h = h_ref[...]
    gi = jax.lax.dot_general(
        a, wih_ref[...], (((1,), (1,)), ((), ())),
        preferred_element_type=jnp.float32) + bih_ref[...]
    gh = jax.lax.dot_general(
        h, whh_ref[...], (((1,), (1,)), ((), ())),
        preferred_element_type=jnp.float32) + bhh_ref[...]
    r = jax.nn.sigmoid(gi[:, :F] + gh[:, :F])
    z = jax.nn.sigmoid(gi[:, F:2 * F] + gh[:, F:2 * F])
    n = jnp.tanh(gi[:, 2 * F:] + r * gh[:, 2 * F:])
    return (1.0 - z) * n + z * h


def _gru_pre_body(ap_ref, h_ref, wih_ref, bih_ref, we_ref, be_ref,
                  whh_ref, bhh_ref, hnew_ref, hcat_ref):
    hn = _gru_math(ap_ref, h_ref, wih_ref, bih_ref, whh_ref, bhh_ref)
    hnew_ref[...] = hn
    for i in range(NT):
        hcat_ref[i] = jax.lax.dot_general(
            hn, we_ref[i], (((1,), (1,)), ((), ())),
            preferred_element_type=jnp.float32) + be_ref[i]


_gru_pre_call = pl.pallas_call(
    _gru_pre_body,
    grid=(GRID,),
    in_specs=[
        pl.BlockSpec((NC, BN, F), lambda i: (0, i, 0)),
        pl.BlockSpec((BN, F), lambda i: (i, 0)),
        pl.BlockSpec((3 * F, F), lambda i: (0, 0)),
        pl.BlockSpec((1, 3 * F), lambda i: (0, 0)),
        pl.BlockSpec((NT, F, F), lambda i: (0, 0, 0)),
        pl.BlockSpec((NT, 1, F), lambda i: (0, 0, 0)),
        pl.BlockSpec((3 * F, F), lambda i: (0, 0)),
        pl.BlockSpec((1, 3 * F), lambda i: (0, 0)),
    ],
    out_specs=[
        pl.BlockSpec((BN, F), lambda i: (i, 0)),
        pl.BlockSpec((NT, BN, F), lambda i: (0, i, 0)),
    ],
    out_shape=[
        jax.ShapeDtypeStruct((NPAD, F), jnp.float32),
        jax.ShapeDtypeStruct((NT, NPAD, F), jnp.float32),
    ],
)


def _gru_head_body(ap_ref, h_ref, wih_ref, bih_ref, whh_ref, bhh_ref,
                   f_ref, w_ref, b_ref, o_ref):
    hn = _gru_math(ap_ref, h_ref, wih_ref, bih_ref, whh_ref, bhh_ref)
    hf = jnp.concatenate([hn, f_ref[...]], axis=1)
    o_ref[...] = jax.nn.sigmoid(
        jax.lax.dot_general(hf, w_ref[...], (((1,), (1,)), ((), ())),
                            preferred_element_type=jnp.float32) + b_ref[0, 0])


_gru_head_call = pl.pallas_call(
    _gru_head_body,
    grid=(GRID,),
    in_specs=[
        pl.BlockSpec((NC, BN, F), lambda i: (0, i, 0)),
        pl.BlockSpec((BN, F), lambda i: (i, 0)),
        pl.BlockSpec((3 * F, F), lambda i: (0, 0)),
        pl.BlockSpec((1, 3 * F), lambda i: (0, 0)),
        pl.BlockSpec((3 * F, F), lambda i: (0, 0)),
        pl.BlockSpec((1, 3 * F), lambda i: (0, 0)),
        pl.BlockSpec((BN, F), lambda i: (i, 0)),
        pl.BlockSpec((8, 2 * F), lambda i: (0, 0)),
        pl.BlockSpec((1, 1), lambda i: (0, 0)),
    ],
    out_specs=pl.BlockSpec((BN, 8), lambda i: (i, 0)),
    out_shape=jax.ShapeDtypeStruct((NPAD, 8), jnp.float32),
)


def _pack_body(s_ref, e_ref, d_ref, p_ref):
    p_ref[...] = (s_ref[...] + e_ref[...] * NPAD) + d_ref[...] * 65536


_pack_call = pl.pallas_call(
    _pack_body,
    in_specs=[pl.BlockSpec((N_EDGES // F, F), lambda: (0, 0))] * 3,
    out_specs=pl.BlockSpec((N_EDGES // F, F), lambda: (0, 0)),
    out_shape=jax.ShapeDtypeStruct((N_EDGES // F, F), jnp.int32),
)


def _sc_agg_body(hcat, packed, out, pk2, rows0, rows1, ci0, ci1, ds0, ds1,
                 acc, sem0, sem1, ses0, ses1):
    cid = jax.lax.axis_index("c")
    sid = jax.lax.axis_index("s")
    wid = cid * NS + sid

    ldp = pltpu.async_copy(packed.at[wid], pk2, sem0)

    zero16 = jnp.zeros((16,), jnp.float32)

    def _zr(i, carry):
        rows0[i // (F // 16), pl.ds((i % (F // 16)) * 16, 16)] = zero16
        return carry

    jax.lax.fori_loop(0, K * (F // 16), _zr, 0)

    def _zacc(kk, carry):
        pltpu.sync_copy(rows0, acc.at[pl.ds(sid * ROWS_PER_TILE + kk * K, K)])
        return carry

    jax.lax.fori_loop(0, ROWS_PER_TILE // K, _zacc, 0)
    ldp.wait()

    cbufs = ((ci0, ds0), (ci1, ds1))

    def _unpack(j, b):
        for q in range(K // 16):
            sl = pl.ds(q * 16, 16)
            p = pk2[j, sl]
            cbufs[b][0][sl] = jnp.bitwise_and(p, 65535)
            cbufs[b][1][sl] = jax.lax.shift_right_logical(p, 16)

    _unpack(0, 0)
    pltpu.async_copy(hcat.at[ci0], rows0, sem0)
    plsc.subcore_barrier()

    bufs = ((rows0, sem0, ses0), (rows1, sem1, ses1))

    def _step(j, b):
        rows, semg, sems = bufs[b]
        nrows, nsemg, nsems = bufs[1 - b]

        @pl.when(jnp.logical_and(j + 1 < NCHUNK, j >= 1))
        def _():
            pltpu.make_async_copy(nrows, acc.at[cbufs[1 - b][1]], nsems).wait()

        @pl.when(j + 1 < NCHUNK)
        def _():
            _unpack(j + 1, 1 - b)
            pltpu.async_copy(hcat.at[cbufs[1 - b][0]], nrows, nsemg)

        pltpu.make_async_copy(hcat.at[cbufs[b][0]], rows, semg).wait()
        pltpu.async_copy(rows, acc.at[cbufs[b][1]], sems, add=True)

    def _pair(jj, carry):
        for b in range(2):
            _step(jj * 2 + b, b)
        return carry

    jax.lax.fori_loop(0, NCHUNK // 2, _pair, 0)
    if NCHUNK % 2:
        _step(NCHUNK - 1, 0)
    pltpu.make_async_copy(rows1, acc.at[ds1], ses1).wait()
    pltpu.make_async_copy(rows0, acc.at[ds0], ses0).wait()

    plsc.subcore_barrier()

    def _wb(kk, carry):
        r0 = sid * ROWS_PER_TILE + kk * K
        pltpu.sync_copy(acc.at[pl.ds(r0, K)], rows0)
        pltpu.sync_copy(rows0, out.at[cid, pl.ds(r0, K)])
        return carry

    jax.lax.fori_loop(0, ROWS_PER_TILE // K, _wb, 0)


@functools.cache
def _sc_agg_call():
    return pl.kernel(
        _sc_agg_body,
        out_type=jax.ShapeDtypeStruct((NC, NPAD, F), jnp.float32),
        mesh=plsc.VectorSubcoreMesh(core_axis_name="c", subcore_axis_name="s",
                                    num_cores=NC, num_subcores=NS),
        scratch_types=[
            pltpu.VMEM((NCHUNK, K), jnp.int32),
            pltpu.VMEM((K, F), jnp.float32),
            pltpu.VMEM((K, F), jnp.float32),
            pltpu.VMEM((K,), jnp.int32),
            pltpu.VMEM((K,), jnp.int32),
            pltpu.VMEM((K,), jnp.int32),
            pltpu.VMEM((K,), jnp.int32),
            pltpu.VMEM_SHARED((NPAD, F), jnp.float32),
            pltpu.SemaphoreType.DMA,
            pltpu.SemaphoreType.DMA,
            pltpu.SemaphoreType.DMA,
            pltpu.SemaphoreType.DMA,
        ],
    )


def kernel(feat, edge_index, etypes, W_e, b_e, W_ih, W_hh, b_ih, b_hh,
           out_W, out_b):
    src = edge_index[0]
    dst = edge_index[1]
    be3 = b_e.reshape(NT, 1, F)
    bhh2 = b_hh.reshape(1, 3 * F)
    bih2 = b_ih.reshape(1, 3 * F)
    ob2 = out_b.reshape(1, 1)
    fpad = jnp.pad(feat, ((0, NPAD - N_NODES), (0, 0)))
    packed = _pack_call(src.reshape(N_EDGES // F, F),
                        etypes.reshape(N_EDGES // F, F),
                        dst.reshape(N_EDGES // F, F)).reshape(NW, NCHUNK, K)

    h = fpad
    hcat = _pre_call(h, W_e, be3)
    for _ in range(NSTEP - 1):
        apart = _sc_agg_call()(hcat.reshape(NT * NPAD, F), packed)
        h, hcat = _gru_pre_call(apart, h, W_ih, bih2, W_e, be3, W_hh, bhh2)
    apart = _sc_agg_call()(hcat.reshape(NT * NPAD, F), packed)
    wpad = jnp.pad(out_W, ((0, 7), (0, 0)))
    out = _gru_head_call(apart, h, W_ih, bih2, W_hh, bhh2, fpad, wpad, ob2)
    return out[:N_NODES, 0]


# speedup vs baseline: 12.7161x; 1.1663x over previous
import functools

import jax
import jax.numpy as jnp
from jax.experimental import pallas as pl
from jax.experimental.pallas import tpu as pltpu
from jax.experimental.pallas import tpu_sc as plsc

F = 128
NT = 3
NSTEP = 3
N_NODES = 10000
N_EDGES = 320000
NPAD = 10240

NC = 2
NS = 16
NW = NC * NS
EPT = N_EDGES // NW
K = 80
NCHUNK = EPT // K
ROWS_PER_TILE = NPAD // NS
BN = 1024
GRID = NPAD // BN


def _pre_body(h_ref, we_ref, be_ref, hcat_ref):
    x = h_ref[...]
    for i in range(NT):
        hcat_ref[i] = jax.lax.dot_general(
            x, we_ref[i], (((1,), (1,)), ((), ())),
            preferred_element_type=jnp.float32) + be_ref[i]


_pre_call = pl.pallas_call(
    _pre_body,
    grid=(GRID,),
    in_specs=[
        pl.BlockSpec((BN, F), lambda i: (i, 0)),
        pl.BlockSpec((NT, F, F), lambda i: (0, 0, 0)),
        pl.BlockSpec((NT, 1, F), lambda i: (0, 0, 0)),
    ],
    out_specs=pl.BlockSpec((NT, BN, F), lambda i: (0, i, 0)),
    out_shape=jax.ShapeDtypeStruct((NT, NPAD, F), jnp.float32),
)


def _gru_math(ap_ref, h_ref, wih_ref, bih_ref, whh_ref, bhh_ref):
    a = ap_ref[0] + ap_ref[1]
    h = h_ref[...]
    gi = jax.lax.dot_general(
        a, wih_ref[...], (((1,), (1,)), ((), ())),
        preferred_element_type=jnp.float32) + bih_ref[...]
    gh = jax.lax.dot_general(
        h, whh_ref[...], (((1,), (1,)), ((), ())),
        preferred_element_type=jnp.float32) + bhh_ref[...]
    r = jax.nn.sigmoid(gi[:, :F] + gh[:, :F])
    z = jax.nn.sigmoid(gi[:, F:2 * F] + gh[:, F:2 * F])
    n = jnp.tanh(gi[:, 2 * F:] + r * gh[:, 2 * F:])
    return (1.0 - z) * n + z * h


def _gru_pre_body(ap_ref, h_ref, wih_ref, bih_ref, we_ref, be_ref,
                  whh_ref, bhh_ref, hnew_ref, hcat_ref):
    hn = _gru_math(ap_ref, h_ref, wih_ref, bih_ref, whh_ref, bhh_ref)
    hnew_ref[...] = hn
    for i in range(NT):
        hcat_ref[i] = jax.lax.dot_general(
            hn, we_ref[i], (((1,), (1,)), ((), ())),
            preferred_element_type=jnp.float32) + be_ref[i]


_gru_pre_call = pl.pallas_call(
    _gru_pre_body,
    grid=(GRID,),
    in_specs=[
        pl.BlockSpec((NC, BN, F), lambda i: (0, i, 0)),
        pl.BlockSpec((BN, F), lambda i: (i, 0)),
        pl.BlockSpec((3 * F, F), lambda i: (0, 0)),
        pl.BlockSpec((1, 3 * F), lambda i: (0, 0)),
        pl.BlockSpec((NT, F, F), lambda i: (0, 0, 0)),
        pl.BlockSpec((NT, 1, F), lambda i: (0, 0, 0)),
        pl.BlockSpec((3 * F, F), lambda i: (0, 0)),
        pl.BlockSpec((1, 3 * F), lambda i: (0, 0)),
    ],
    out_specs=[
        pl.BlockSpec((BN, F), lambda i: (i, 0)),
        pl.BlockSpec((NT, BN, F), lambda i: (0, i, 0)),
    ],
    out_shape=[
        jax.ShapeDtypeStruct((NPAD, F), jnp.float32),
        jax.ShapeDtypeStruct((NT, NPAD, F), jnp.float32),
    ],
)


def _gru_head_body(ap_ref, h_ref, wih_ref, bih_ref, whh_ref, bhh_ref,
                   f_ref, w_ref, b_ref, o_ref):
    hn = _gru_math(ap_ref, h_ref, wih_ref, bih_ref, whh_ref, bhh_ref)
    hf = jnp.concatenate([hn, f_ref[...]], axis=1)
    o_ref[...] = jax.nn.sigmoid(
        jax.lax.dot_general(hf, w_ref[...], (((1,), (1,)), ((), ())),
                            preferred_element_type=jnp.float32) + b_ref[0, 0])


_gru_head_call = pl.pallas_call(
    _gru_head_body,
    grid=(GRID,),
    in_specs=[
        pl.BlockSpec((NC, BN, F), lambda i: (0, i, 0)),
        pl.BlockSpec((BN, F), lambda i: (i, 0)),
        pl.BlockSpec((3 * F, F), lambda i: (0, 0)),
        pl.BlockSpec((1, 3 * F), lambda i: (0, 0)),
        pl.BlockSpec((3 * F, F), lambda i: (0, 0)),
        pl.BlockSpec((1, 3 * F), lambda i: (0, 0)),
        pl.BlockSpec((BN, F), lambda i: (i, 0)),
        pl.BlockSpec((8, 2 * F), lambda i: (0, 0)),
        pl.BlockSpec((1, 1), lambda i: (0, 0)),
    ],
    out_specs=pl.BlockSpec((BN, 8), lambda i: (i, 0)),
    out_shape=jax.ShapeDtypeStruct((NPAD, 8), jnp.float32),
)


def _pack_body(s_ref, e_ref, d_ref, p_ref):
    p_ref[...] = (s_ref[...] + e_ref[...] * NPAD) + d_ref[...] * 65536


_pack_call = pl.pallas_call(
    _pack_body,
    in_specs=[pl.BlockSpec((N_EDGES // F, F), lambda: (0, 0))] * 3,
    out_specs=pl.BlockSpec((N_EDGES // F, F), lambda: (0, 0)),
    out_shape=jax.ShapeDtypeStruct((N_EDGES // F, F), jnp.int32),
)


def _sc_agg_body(hcat, packed, out, pk2, rows0, rows1, ci0, ci1, ds0, ds1,
                 acc, sem0, sem1, ses0, ses1):
    cid = jax.lax.axis_index("c")
    sid = jax.lax.axis_index("s")
    wid = cid * NS + sid

    ldp = pltpu.async_copy(packed.at[wid], pk2, sem0)

    zero16 = jnp.zeros((16,), jnp.float32)

    def _zr(i, carry):
        rows0[i // (F // 16), pl.ds((i % (F // 16)) * 16, 16)] = zero16
        return carry

    jax.lax.fori_loop(0, K * (F // 16), _zr, 0)

    def _zacc(kk, carry):
        pltpu.sync_copy(rows0, acc.at[pl.ds(sid * ROWS_PER_TILE + kk * K, K)])
        return carry

    jax.lax.fori_loop(0, ROWS_PER_TILE // K, _zacc, 0)
    ldp.wait()

    cbufs = ((ci0, ds0), (ci1, ds1))

    def _unpack(j, b):
        for q in range(K // 16):
            sl = pl.ds(q * 16, 16)
            p = pk2[j, sl]
            cbufs[b][0][sl] = jnp.bitwise_and(p, 65535)
            cbufs[b][1][sl] = jax.lax.shift_right_logical(p, 16)

    _unpack(0, 0)
    pltpu.async_copy(hcat.at[ci0], rows0, sem0)
    plsc.subcore_barrier()

    bufs = ((rows0, sem0, ses0), (rows1, sem1, ses1))

    def _step(j, b):
        rows, semg, sems = bufs[b]
        nrows, nsemg, nsems = bufs[1 - b]

        @pl.when(jnp.logical_and(j + 1 < NCHUNK, j >= 1))
        def _():
            pltpu.make_async_copy(nrows, acc.at[cbufs[1 - b][1]], nsems).wait()

        @pl.when(j + 1 < NCHUNK)
        def _():
            _unpack(j + 1, 1 - b)
            pltpu.async_copy(hcat.at[cbufs[1 - b][0]], nrows, nsemg)

        pltpu.make_async_copy(hcat.at[cbufs[b][0]], rows, semg).wait()
        pltpu.async_copy(rows, acc.at[cbufs[b][1]], sems, add=True)

    def _pair(jj, carry):
        for b in range(2):
            _step(jj * 2 + b, b)
        return carry

    jax.lax.fori_loop(0, NCHUNK // 2, _pair, 0)
    if NCHUNK % 2:
        _step(NCHUNK - 1, 0)
    pltpu.make_async_copy(rows1, acc.at[ds1], ses1).wait()
    pltpu.make_async_copy(rows0, acc.at[ds0], ses0).wait()

    plsc.subcore_barrier()

    r0 = sid * ROWS_PER_TILE
    pltpu.sync_copy(acc.at[pl.ds(r0, ROWS_PER_TILE)],
                    out.at[cid, pl.ds(r0, ROWS_PER_TILE)])


@functools.cache
def _sc_agg_call():
    return pl.kernel(
        _sc_agg_body,
        out_type=jax.ShapeDtypeStruct((NC, NPAD, F), jnp.float32),
        mesh=plsc.VectorSubcoreMesh(core_axis_name="c", subcore_axis_name="s",
                                    num_cores=NC, num_subcores=NS),
        scratch_types=[
            pltpu.VMEM((NCHUNK, K), jnp.int32),
            pltpu.VMEM((K, F), jnp.float32),
            pltpu.VMEM((K, F), jnp.float32),
            pltpu.VMEM((K,), jnp.int32),
            pltpu.VMEM((K,), jnp.int32),
            pltpu.VMEM((K,), jnp.int32),
            pltpu.VMEM((K,), jnp.int32),
            pltpu.VMEM_SHARED((NPAD, F), jnp.float32),
            pltpu.SemaphoreType.DMA,
            pltpu.SemaphoreType.DMA,
            pltpu.SemaphoreType.DMA,
            pltpu.SemaphoreType.DMA,
        ],
    )


def kernel(feat, edge_index, etypes, W_e, b_e, W_ih, W_hh, b_ih, b_hh,
           out_W, out_b):
    src = edge_index[0]
    dst = edge_index[1]
    be3 = b_e.reshape(NT, 1, F)
    bhh2 = b_hh.reshape(1, 3 * F)
    bih2 = b_ih.reshape(1, 3 * F)
    ob2 = out_b.reshape(1, 1)
    fpad = jnp.pad(feat, ((0, NPAD - N_NODES), (0, 0)))
    packed = _pack_call(src.reshape(N_EDGES // F, F),
                        etypes.reshape(N_EDGES // F, F),
                        dst.reshape(N_EDGES // F, F)).reshape(NW, NCHUNK, K)

    h = fpad
    hcat = _pre_call(h, W_e, be3)
    for _ in range(NSTEP - 1):
        apart = _sc_agg_call()(hcat.reshape(NT * NPAD, F), packed)
        h, hcat = _gru_pre_call(apart, h, W_ih, bih2, W_e, be3, W_hh, bhh2)
    apart = _sc_agg_call()(hcat.reshape(NT * NPAD, F), packed)
    wpad = jnp.pad(out_W, ((0, 7), (0, 0)))
    out = _gru_head_call(apart, h, W_ih, bih2, W_hh, bhh2, fpad, wpad, ob2)
    return out[:N_NODES, 0]


# speedup vs baseline: 12.7240x; 1.0006x over previous
import functools

import jax
import jax.numpy as jnp
from jax.experimental import pallas as pl
from jax.experimental.pallas import tpu as pltpu
from jax.experimental.pallas import tpu_sc as plsc

F = 128
NT = 3
NSTEP = 3
N_NODES = 10000
N_EDGES = 320000
NPAD = 10240

NC = 2
NS = 16
NW = NC * NS
EPT = N_EDGES // NW
K = 80
NCHUNK = EPT // K
ROWS_PER_TILE = NPAD // NS
BN = 1024
GRID = NPAD // BN


EROWS = 2560
EB = EROWS // GRID


def _pre_body(h_ref, we_ref, be_ref, s_ref, e_ref, d_ref, hcat_ref, p_ref):
    x = h_ref[...]
    for i in range(NT):
        hcat_ref[i] = jax.lax.dot_general(
            x, we_ref[i], (((1,), (1,)), ((), ())),
            preferred_element_type=jnp.float32) + be_ref[i]
    p_ref[...] = (s_ref[...] + e_ref[...] * NPAD) + d_ref[...] * 65536


_pre_call = pl.pallas_call(
    _pre_body,
    grid=(GRID,),
    in_specs=[
        pl.BlockSpec((BN, F), lambda i: (i, 0)),
        pl.BlockSpec((NT, F, F), lambda i: (0, 0, 0)),
        pl.BlockSpec((NT, 1, F), lambda i: (0, 0, 0)),
        pl.BlockSpec((EB, F), lambda i: (i, 0)),
        pl.BlockSpec((EB, F), lambda i: (i, 0)),
        pl.BlockSpec((EB, F), lambda i: (i, 0)),
    ],
    out_specs=[
        pl.BlockSpec((NT, BN, F), lambda i: (0, i, 0)),
        pl.BlockSpec((EB, F), lambda i: (i, 0)),
    ],
    out_shape=[
        jax.ShapeDtypeStruct((NT, NPAD, F), jnp.float32),
        jax.ShapeDtypeStruct((EROWS, F), jnp.int32),
    ],
)


def _gru_math(ap_ref, h_ref, wih_ref, bih_ref, whh_ref, bhh_ref):
    a = ap_ref[0] + ap_ref[1]
    h = h_ref[...]
    gi = jax.lax.dot_general(
        a, wih_ref[...], (((1,), (1,)), ((), ())),
        preferred_element_type=jnp.float32) + bih_ref[...]
    gh = jax.lax.dot_general(
        h, whh_ref[...], (((1,), (1,)), ((), ())),
        preferred_element_type=jnp.float32) + bhh_ref[...]
    r = jax.nn.sigmoid(gi[:, :F] + gh[:, :F])
    z = jax.nn.sigmoid(gi[:, F:2 * F] + gh[:, F:2 * F])
    n = jnp.tanh(gi[:, 2 * F:] + r * gh[:, 2 * F:])
    return (1.0 - z) * n + z * h


def _gru_pre_body(ap_ref, h_ref, wih_ref, bih_ref, we_ref, be_ref,
                  whh_ref, bhh_ref, hnew_ref, hcat_ref):
    hn = _gru_math(ap_ref, h_ref, wih_ref, bih_ref, whh_ref, bhh_ref)
    hnew_ref[...] = hn
    for i in range(NT):
        hcat_ref[i] = jax.lax.dot_general(
            hn, we_ref[i], (((1,), (1,)), ((), ())),
            preferred_element_type=jnp.float32) + be_ref[i]


_gru_pre_call = pl.pallas_call(
    _gru_pre_body,
    grid=(GRID,),
    in_specs=[
        pl.BlockSpec((NC, BN, F), lambda i: (0, i, 0)),
        pl.BlockSpec((BN, F), lambda i: (i, 0)),
        pl.BlockSpec((3 * F, F), lambda i: (0, 0)),
        pl.BlockSpec((1, 3 * F), lambda i: (0, 0)),
        pl.BlockSpec((NT, F, F), lambda i: (0, 0, 0)),
        pl.BlockSpec((NT, 1, F), lambda i: (0, 0, 0)),
        pl.BlockSpec((3 * F, F), lambda i: (0, 0)),
        pl.BlockSpec((1, 3 * F), lambda i: (0, 0)),
    ],
    out_specs=[
        pl.BlockSpec((BN, F), lambda i: (i, 0)),
        pl.BlockSpec((NT, BN, F), lambda i: (0, i, 0)),
    ],
    out_shape=[
        jax.ShapeDtypeStruct((NPAD, F), jnp.float32),
        jax.ShapeDtypeStruct((NT, NPAD, F), jnp.float32),
    ],
)


def _gru_head_body(ap_ref, h_ref, wih_ref, bih_ref, whh_ref, bhh_ref,
                   f_ref, w_ref, b_ref, o_ref):
    hn = _gru_math(ap_ref, h_ref, wih_ref, bih_ref, whh_ref, bhh_ref)
    hf = jnp.concatenate([hn, f_ref[...]], axis=1)
    o_ref[...] = jax.nn.sigmoid(
        jax.lax.dot_general(hf, w_ref[...], (((1,), (1,)), ((), ())),
                            preferred_element_type=jnp.float32) + b_ref[0, 0])


_gru_head_call = pl.pallas_call(
    _gru_head_body,
    grid=(GRID,),
    in_specs=[
        pl.BlockSpec((NC, BN, F), lambda i: (0, i, 0)),
        pl.BlockSpec((BN, F), lambda i: (i, 0)),
        pl.BlockSpec((3 * F, F), lambda i: (0, 0)),
        pl.BlockSpec((1, 3 * F), lambda i: (0, 0)),
        pl.BlockSpec((3 * F, F), lambda i: (0, 0)),
        pl.BlockSpec((1, 3 * F), lambda i: (0, 0)),
        pl.BlockSpec((BN, F), lambda i: (i, 0)),
        pl.BlockSpec((8, 2 * F), lambda i: (0, 0)),
        pl.BlockSpec((1, 1), lambda i: (0, 0)),
    ],
    out_specs=pl.BlockSpec((BN, 8), lambda i: (i, 0)),
    out_shape=jax.ShapeDtypeStruct((NPAD, 8), jnp.float32),
)


def _sc_agg_body(hcat, packed, out, pk2, rows0, rows1, ci0, ci1, ds0, ds1,
                 acc, sem0, sem1, ses0, ses1):
    cid = jax.lax.axis_index("c")
    sid = jax.lax.axis_index("s")
    wid = cid * NS + sid

    ldp = pltpu.async_copy(packed.at[wid], pk2, sem0)

    zero16 = jnp.zeros((16,), jnp.float32)

    def _zr(i, carry):
        rows0[i // (F // 16), pl.ds((i % (F // 16)) * 16, 16)] = zero16
        return carry

    jax.lax.fori_loop(0, K * (F // 16), _zr, 0)

    def _zacc(kk, carry):
        pltpu.sync_copy(rows0, acc.at[pl.ds(sid * ROWS_PER_TILE + kk * K, K)])
        return carry

    jax.lax.fori_loop(0, ROWS_PER_TILE // K, _zacc, 0)
    ldp.wait()

    cbufs = ((ci0, ds0), (ci1, ds1))

    def _unpack(j, b):
        for q in range(K // 16):
            sl = pl.ds(q * 16, 16)
            p = pk2[j, sl]
            cbufs[b][0][sl] = jnp.bitwise_and(p, 65535)
            cbufs[b][1][sl] = jax.lax.shift_right_logical(p, 16)

    _unpack(0, 0)
    pltpu.async_copy(hcat.at[ci0], rows0, sem0)
    plsc.subcore_barrier()

    bufs = ((rows0, sem0, ses0), (rows1, sem1, ses1))

    def _step(j, b):
        rows, semg, sems = bufs[b]
        nrows, nsemg, nsems = bufs[1 - b]

        @pl.when(jnp.logical_and(j + 1 < NCHUNK, j >= 1))
        def _():
            pltpu.make_async_copy(nrows, acc.at[cbufs[1 - b][1]], nsems).wait()

        @pl.when(j + 1 < NCHUNK)
        def _():
            _unpack(j + 1, 1 - b)
            pltpu.async_copy(hcat.at[cbufs[1 - b][0]], nrows, nsemg)

        pltpu.make_async_copy(hcat.at[cbufs[b][0]], rows, semg).wait()
        pltpu.async_copy(rows, acc.at[cbufs[b][1]], sems, add=True)

    def _pair(jj, carry):
        for b in range(2):
            _step(jj * 2 + b, b)
        return carry

    jax.lax.fori_loop(0, NCHUNK // 2, _pair, 0)
    if NCHUNK % 2:
        _step(NCHUNK - 1, 0)
    pltpu.make_async_copy(rows1, acc.at[ds1], ses1).wait()
    pltpu.make_async_copy(rows0, acc.at[ds0], ses0).wait()

    plsc.subcore_barrier()

    r0 = sid * ROWS_PER_TILE
    pltpu.sync_copy(acc.at[pl.ds(r0, ROWS_PER_TILE)],
                    out.at[cid, pl.ds(r0, ROWS_PER_TILE)])


@functools.cache
def _sc_agg_call():
    return pl.kernel(
        _sc_agg_body,
        out_type=jax.ShapeDtypeStruct((NC, NPAD, F), jnp.float32),
        mesh=plsc.VectorSubcoreMesh(core_axis_name="c", subcore_axis_name="s",
                                    num_cores=NC, num_subcores=NS),
        scratch_types=[
            pltpu.VMEM((NCHUNK, K), jnp.int32),
            pltpu.VMEM((K, F), jnp.float32),
            pltpu.VMEM((K, F), jnp.float32),
            pltpu.VMEM((K,), jnp.int32),
            pltpu.VMEM((K,), jnp.int32),
            pltpu.VMEM((K,), jnp.int32),
            pltpu.VMEM((K,), jnp.int32),
            pltpu.VMEM_SHARED((NPAD, F), jnp.float32),
            pltpu.SemaphoreType.DMA,
            pltpu.SemaphoreType.DMA,
            pltpu.SemaphoreType.DMA,
            pltpu.SemaphoreType.DMA,
        ],
    )


def kernel(feat, edge_index, etypes, W_e, b_e, W_ih, W_hh, b_ih, b_hh,
           out_W, out_b):
    src = edge_index[0]
    dst = edge_index[1]
    be3 = b_e.reshape(NT, 1, F)
    bhh2 = b_hh.reshape(1, 3 * F)
    bih2 = b_ih.reshape(1, 3 * F)
    ob2 = out_b.reshape(1, 1)
    fpad = jnp.pad(feat, ((0, NPAD - N_NODES), (0, 0)))
    epad = EROWS * F - N_EDGES
    spad = jnp.pad(src, (0, epad)).reshape(EROWS, F)
    tpad = jnp.pad(etypes, (0, epad)).reshape(EROWS, F)
    dpad = jnp.pad(dst, (0, epad)).reshape(EROWS, F)

    h = fpad
    hcat, packed = _pre_call(h, W_e, be3, spad, tpad, dpad)
    packed = packed.reshape(-1)[:N_EDGES].reshape(NW, NCHUNK, K)
    for _ in range(NSTEP - 1):
        apart = _sc_agg_call()(hcat.reshape(NT * NPAD, F), packed)
        h, hcat = _gru_pre_call(apart, h, W_ih, bih2, W_e, be3, W_hh, bhh2)
    apart = _sc_agg_call()(hcat.reshape(NT * NPAD, F), packed)
    wpad = jnp.pad(out_W, ((0, 7), (0, 0)))
    out = _gru_head_call(apart, h, W_ih, bih2, W_hh, bhh2, fpad, wpad, ob2)
    return out[:N_NODES, 0]
